# speedup vs baseline: 21.1022x; 21.1022x over previous
import jax
import jax.numpy as jnp
from jax.experimental import pallas as pl
from jax.experimental.pallas import tpu as pltpu

_B = 4096
_S = 200
_H = 128
_BB = 128


def _tc_body(idx_ref, emb_int_ref, emb_pos_ref, out_ref):
    idx = idx_ref[...]
    pos = emb_pos_ref[...]
    c0 = (pos + emb_int_ref[0, :][None, :])[None, :, :]
    c1 = (pos + emb_int_ref[1, :][None, :])[None, :, :]
    c2 = (pos + emb_int_ref[2, :][None, :])[None, :, :]
    m = idx[:, :, None]
    out_ref[...] = jnp.where(m == 0, c0, jnp.where(m == 1, c1, c2))


def kernel(interaction, emb_interaction, emb_position):
    grid = (_B // _BB,)
    return pl.pallas_call(
        _tc_body,
        grid=grid,
        in_specs=[
            pl.BlockSpec((_BB, _S), lambda i: (i, 0)),
            pl.BlockSpec((3, _H), lambda i: (0, 0)),
            pl.BlockSpec((_S, _H), lambda i: (0, 0)),
        ],
        out_specs=pl.BlockSpec((_BB, _S, _H), lambda i: (i, 0, 0)),
        out_shape=jax.ShapeDtypeStruct((_B, _S, _H), jnp.float32),
    )(interaction, emb_interaction, emb_position)
